# baseline (speedup 1.0000x reference)
import functools

import jax
import jax.numpy as jnp
import numpy as np
from jax import lax
from jax.experimental import pallas as pl
from jax.experimental.pallas import tpu as pltpu
from jax.experimental.pallas import tpu_sc as plsc

VOCAB = 100000
DIM = 64
B = 4096
L = 200

_info = plsc.get_sparse_core_info()
NC, NS = _info.num_cores, _info.num_subcores
NW = NC * NS
SEQ_W = B // NW
PBLK = 2
SQRT = 128
NQ_U = SEQ_W // SQRT
N_UNITS = (L // PBLK) * NQ_U
ROWS_U = PBLK * SQRT
GPU_ = ROWS_U // 128
SQP = SQRT + 1


def _pos_encoding_np():
    positions = np.arange(L)[:, np.newaxis].astype(np.float64)
    dims = np.arange(DIM)[np.newaxis, :].astype(np.float64)
    angles = positions / np.power(10000.0, 2 * (dims // 2) / DIM)
    angles[:, 0::2] = np.sin(angles[:, 0::2])
    angles[:, 1::2] = np.cos(angles[:, 1::2])
    return angles.astype(np.float32)


_POS_NP = _pos_encoding_np()


def _bias_body(pos_ref, beat_ref, bar_ref, out_ref):
    beat = jnp.tile(beat_ref[...], (L // 4, 1))
    bar = jnp.tile(bar_ref[...], (L // 16 + 1, 1))[:L]
    out_ref[...] = pos_ref[...] + beat + bar


def _sc_body(xu_hbm, tok_hbm, bias_hbm, out_hbm,
             idx_v, bias_v, g0, g1, p0b, p1b, sg0, sg1, ss0, ss1):
    cid = lax.axis_index("c")
    sid = lax.axis_index("s")
    wid = sid * NC + cid
    lane0 = wid * SEQ_W

    gs = (g0, g1)
    ps = (p0b, p1b)
    sgs = (sg0, sg1)
    sss = (ss0, ss1)

    pltpu.sync_copy(xu_hbm.at[wid], idx_v)
    pltpu.sync_copy(bias_hbm, bias_v)

    pd_vecs = [[lax.iota(jnp.int32, 16) + (p * DIM + 16 * dd)
                for dd in range(DIM // 16)] for p in range(PBLK)]

    def unit_pos(u):
        return lax.div(u, NQ_U) * PBLK, lax.rem(u, NQ_U) * SQRT

    def fire_gathers(u, b):
        for k in range(GPU_):
            pltpu.async_copy(
                tok_hbm.at[idx_v.at[u, k]],
                gs[b].at[pl.ds(k * 128, 128)],
                sgs[b])

    def drain_gathers(b):
        for k in range(GPU_):
            pltpu.make_async_copy(
                tok_hbm.at[idx_v.at[0, k]],
                gs[b].at[pl.ds(k * 128, 128)], sgs[b]).wait()

    def fire_store(u, b):
        pp0, ss0_ = unit_pos(u)
        for p in range(PBLK):
            pltpu.async_copy(
                ps[b].at[pl.ds(p * DIM, DIM), pl.ds(0, SQRT)],
                out_hbm.at[pp0 + p, :, pl.ds(lane0 + ss0_, SQRT)],
                sss[b])

    def drain_store(b):
        for p in range(PBLK):
            pltpu.make_async_copy(
                ps[b].at[pl.ds(p * DIM, DIM), pl.ds(0, SQRT)],
                out_hbm.at[p, :, pl.ds(lane0, SQRT)],
                sss[b]).wait()

    def transpose_unit(u, b):
        pp0, _ = unit_pos(u)
        gb, pb = gs[b], ps[b]
        for p in range(PBLK):
            brow = [bias_v[pp0 + p, pl.ds(16 * dd, 16)]
                    for dd in range(DIM // 16)]

            def srow(s, s_vec):
                r = p * SQRT + s
                vals = [gb[r, pl.ds(16 * dd, 16)] + brow[dd]
                        for dd in range(DIM // 16)]
                for dd in range(DIM // 16):
                    plsc.store_scatter(pb, [pd_vecs[p][dd], s_vec], vals[dd])
                return s_vec + 1

            lax.fori_loop(0, SQRT, srow,
                          jnp.zeros((16,), jnp.int32), unroll=8)

    fire_gathers(0, 0)
    fire_gathers(1, 1)

    def body(tt, _):
        for b in range(2):
            u = 2 * tt + b
            drain_gathers(b)

            @pl.when(tt >= 1)
            def _ds():
                drain_store(b)

            transpose_unit(u, b)
            fire_store(u, b)

            @pl.when(u + 2 < N_UNITS)
            def _fg():
                fire_gathers(u + 2, b)

        return _

    lax.fori_loop(0, N_UNITS // 2, body, 0, unroll=False)
    drain_store(0)
    drain_store(1)


@jax.jit
def _run(xu, token_table, beat_table, bar_table):
    pos = jnp.asarray(_POS_NP)
    bias = pl.pallas_call(
        _bias_body,
        out_shape=jax.ShapeDtypeStruct((L, DIM), jnp.float32),
    )(pos, beat_table, bar_table)

    mesh = plsc.VectorSubcoreMesh(core_axis_name="c", subcore_axis_name="s")
    f = pl.kernel(
        _sc_body,
        out_type=jax.ShapeDtypeStruct((L, DIM, B), jnp.float32),
        mesh=mesh,
        scratch_types=[
            pltpu.VMEM((N_UNITS, GPU_, 128), jnp.int32),
            pltpu.VMEM((L, DIM), jnp.float32),
            pltpu.VMEM((ROWS_U, DIM), jnp.float32),
            pltpu.VMEM((ROWS_U, DIM), jnp.float32),
            pltpu.VMEM((PBLK * DIM, SQP), jnp.float32),
            pltpu.VMEM((PBLK * DIM, SQP), jnp.float32),
            pltpu.SemaphoreType.DMA,
            pltpu.SemaphoreType.DMA,
            pltpu.SemaphoreType.DMA,
            pltpu.SemaphoreType.DMA,
        ],
        compiler_params=pltpu.CompilerParams(use_tc_tiling_on_sc=False,
                                             needs_layout_passes=False),
        name="rhythm_embed_sc",
    )
    out_t = f(xu, token_table, bias)
    return jnp.transpose(out_t, (2, 0, 1))


def kernel(x, token_table, beat_table, bar_table):
    xi = x.astype(jnp.int32)
    xu = (xi.reshape(NW, SQRT, L // PBLK, PBLK)
          .transpose(0, 2, 3, 1)
          .reshape(NW, N_UNITS, GPU_, 128))
    return _run(xu, token_table, beat_table, bar_table)


# speedup vs baseline: 1.0023x; 1.0023x over previous
import functools

import jax
import jax.numpy as jnp
import numpy as np
from jax import lax
from jax.experimental import pallas as pl
from jax.experimental.pallas import tpu as pltpu
from jax.experimental.pallas import tpu_sc as plsc

VOCAB = 100000
DIM = 64
B = 4096
L = 200

_info = plsc.get_sparse_core_info()
NC, NS = _info.num_cores, _info.num_subcores
NW = NC * NS
SEQ_W = B // NW
PBLK = 2
SQRT = 128
NQ_U = SEQ_W // SQRT
N_UNITS = (L // PBLK) * NQ_U
ROWS_U = PBLK * SQRT
GPU_ = ROWS_U // 128
SQP = SQRT + 1


def _pos_encoding_np():
    positions = np.arange(L)[:, np.newaxis].astype(np.float64)
    dims = np.arange(DIM)[np.newaxis, :].astype(np.float64)
    angles = positions / np.power(10000.0, 2 * (dims // 2) / DIM)
    angles[:, 0::2] = np.sin(angles[:, 0::2])
    angles[:, 1::2] = np.cos(angles[:, 1::2])
    return angles.astype(np.float32)


_POS_NP = _pos_encoding_np()


def _bias_body(pos_ref, beat_ref, bar_ref, out_ref):
    beat = jnp.tile(beat_ref[...], (L // 4, 1))
    bar = jnp.tile(bar_ref[...], (L // 16 + 1, 1))[:L]
    out_ref[...] = pos_ref[...] + beat + bar


def _sc_body(xu_hbm, tok_hbm, bias_hbm, out_hbm,
             idx_v, bias_v, g0, g1, p0b, p1b,
             sg00, sg01, sg10, sg11, ss0, ss1):
    cid = lax.axis_index("c")
    sid = lax.axis_index("s")
    wid = sid * NC + cid
    lane0 = wid * SEQ_W

    gs = (g0, g1)
    ps = (p0b, p1b)
    sgs = ((sg00, sg01), (sg10, sg11))
    sss = (ss0, ss1)

    pltpu.sync_copy(xu_hbm.at[wid], idx_v)
    pltpu.sync_copy(bias_hbm, bias_v)

    pd_vecs = [[lax.iota(jnp.int32, 16) + (p * DIM + 16 * dd)
                for dd in range(DIM // 16)] for p in range(PBLK)]

    def unit_pos(u):
        return lax.div(u, NQ_U) * PBLK, lax.rem(u, NQ_U) * SQRT

    def fire_gather(u, b, k):
        pltpu.async_copy(
            tok_hbm.at[idx_v.at[u, k]],
            gs[b].at[pl.ds(k * 128, 128)],
            sgs[b][k])

    def drain_gather(b, k):
        pltpu.make_async_copy(
            tok_hbm.at[idx_v.at[0, k]],
            gs[b].at[pl.ds(k * 128, 128)], sgs[b][k]).wait()

    def fire_store(u, b, p):
        pp0, ss0_ = unit_pos(u)
        pltpu.async_copy(
            ps[b].at[pl.ds(p * DIM, DIM), pl.ds(0, SQRT)],
            out_hbm.at[pp0 + p, :, pl.ds(lane0 + ss0_, SQRT)],
            sss[b])

    def drain_stores(b):
        for p in range(PBLK):
            pltpu.make_async_copy(
                ps[b].at[pl.ds(p * DIM, DIM), pl.ds(0, SQRT)],
                out_hbm.at[p, :, pl.ds(lane0, SQRT)],
                sss[b]).wait()

    def transpose_p(u, b, p):
        pp0, _ = unit_pos(u)
        gb, pb = gs[b], ps[b]
        brow = [bias_v[pp0 + p, pl.ds(16 * dd, 16)]
                for dd in range(DIM // 16)]

        def srow(s, s_vec):
            r = p * SQRT + s
            vals = [gb[r, pl.ds(16 * dd, 16)] + brow[dd]
                    for dd in range(DIM // 16)]
            for dd in range(DIM // 16):
                plsc.store_scatter(pb, [pd_vecs[p][dd], s_vec], vals[dd])
            return s_vec + 1

        lax.fori_loop(0, SQRT, srow,
                      jnp.zeros((16,), jnp.int32), unroll=4)

    for b in range(2):
        for k in range(GPU_):
            fire_gather(b, b, k)

    def body(tt, _):
        for b in range(2):
            u = 2 * tt + b
            drain_gather(b, 0)

            @pl.when(tt >= 1)
            def _ds():
                drain_stores(b)

            transpose_p(u, b, 0)
            fire_store(u, b, 0)

            @pl.when(u + 2 < N_UNITS)
            def _fg0():
                fire_gather(u + 2, b, 0)

            drain_gather(b, 1)
            transpose_p(u, b, 1)
            fire_store(u, b, 1)

            @pl.when(u + 2 < N_UNITS)
            def _fg1():
                fire_gather(u + 2, b, 1)

        return _

    lax.fori_loop(0, N_UNITS // 2, body, 0, unroll=False)
    drain_stores(0)
    drain_stores(1)


@jax.jit
def _run(xu, token_table, beat_table, bar_table):
    pos = jnp.asarray(_POS_NP)
    bias = pl.pallas_call(
        _bias_body,
        out_shape=jax.ShapeDtypeStruct((L, DIM), jnp.float32),
    )(pos, beat_table, bar_table)

    mesh = plsc.VectorSubcoreMesh(core_axis_name="c", subcore_axis_name="s")
    f = pl.kernel(
        _sc_body,
        out_type=jax.ShapeDtypeStruct((L, DIM, B), jnp.float32),
        mesh=mesh,
        scratch_types=[
            pltpu.VMEM((N_UNITS, GPU_, 128), jnp.int32),
            pltpu.VMEM((L, DIM), jnp.float32),
            pltpu.VMEM((ROWS_U, DIM), jnp.float32),
            pltpu.VMEM((ROWS_U, DIM), jnp.float32),
            pltpu.VMEM((PBLK * DIM, SQP), jnp.float32),
            pltpu.VMEM((PBLK * DIM, SQP), jnp.float32),
            pltpu.SemaphoreType.DMA,
            pltpu.SemaphoreType.DMA,
            pltpu.SemaphoreType.DMA,
            pltpu.SemaphoreType.DMA,
            pltpu.SemaphoreType.DMA,
            pltpu.SemaphoreType.DMA,
        ],
        compiler_params=pltpu.CompilerParams(use_tc_tiling_on_sc=False,
                                             needs_layout_passes=False),
        name="rhythm_embed_sc",
    )
    out_t = f(xu, token_table, bias)
    return jnp.transpose(out_t, (2, 0, 1))


def kernel(x, token_table, beat_table, bar_table):
    xi = x.astype(jnp.int32)
    xu = (xi.reshape(NW, SQRT, L // PBLK, PBLK)
          .transpose(0, 2, 3, 1)
          .reshape(NW, N_UNITS, GPU_, 128))
    return _run(xu, token_table, beat_table, bar_table)


# speedup vs baseline: 1.0033x; 1.0011x over previous
import functools

import jax
import jax.numpy as jnp
import numpy as np
from jax import lax
from jax.experimental import pallas as pl
from jax.experimental.pallas import tpu as pltpu
from jax.experimental.pallas import tpu_sc as plsc

VOCAB = 100000
DIM = 64
B = 4096
L = 200

_info = plsc.get_sparse_core_info()
NC, NS = _info.num_cores, _info.num_subcores
NW = NC * NS
SEQ_W = B // NW
PBLK = 2
SQRT = 128
NQ_U = SEQ_W // SQRT
N_UNITS = (L // PBLK) * NQ_U
ROWS_U = PBLK * SQRT
GPU_ = ROWS_U // 128
SQP = SQRT + 1


def _pos_encoding_np():
    positions = np.arange(L)[:, np.newaxis].astype(np.float64)
    dims = np.arange(DIM)[np.newaxis, :].astype(np.float64)
    angles = positions / np.power(10000.0, 2 * (dims // 2) / DIM)
    angles[:, 0::2] = np.sin(angles[:, 0::2])
    angles[:, 1::2] = np.cos(angles[:, 1::2])
    return angles.astype(np.float32)


_POS_NP = _pos_encoding_np()


def _bias_body(pos_ref, beat_ref, bar_ref, out_ref):
    beat = jnp.tile(beat_ref[...], (L // 4, 1))
    bar = jnp.tile(bar_ref[...], (L // 16 + 1, 1))[:L]
    out_ref[...] = pos_ref[...] + beat + bar


def _sc_body(xu_hbm, tok_hbm, bias_hbm, out_hbm,
             idx_v, bias_v, g0, g1, p0b, p1b,
             sg00, sg01, sg10, sg11, ss0, ss1):
    cid = lax.axis_index("c")
    sid = lax.axis_index("s")
    wid = sid * NC + cid
    lane0 = wid * SEQ_W

    gs = (g0, g1)
    ps = (p0b, p1b)
    sgs = ((sg00, sg01), (sg10, sg11))
    sss = (ss0, ss1)

    pltpu.sync_copy(xu_hbm.at[wid], idx_v)
    pltpu.sync_copy(bias_hbm, bias_v)

    pd_vecs = [[lax.iota(jnp.int32, 16) + (p * DIM + 16 * dd)
                for dd in range(DIM // 16)] for p in range(PBLK)]

    def unit_pos(u):
        return lax.div(u, NQ_U) * PBLK, lax.rem(u, NQ_U) * SQRT

    def fire_gather(u, b, k):
        pltpu.async_copy(
            tok_hbm.at[idx_v.at[u, k]],
            gs[b].at[pl.ds(k * 128, 128)],
            sgs[b][k])

    def drain_gather(b, k):
        pltpu.make_async_copy(
            tok_hbm.at[idx_v.at[0, k]],
            gs[b].at[pl.ds(k * 128, 128)], sgs[b][k]).wait()

    def fire_store(u, b, p):
        pp0, ss0_ = unit_pos(u)
        pltpu.async_copy(
            ps[b].at[pl.ds(p * DIM, DIM), pl.ds(0, SQRT)],
            out_hbm.at[pp0 + p, :, pl.ds(lane0 + ss0_, SQRT)],
            sss[b])

    def drain_stores(b):
        for p in range(PBLK):
            pltpu.make_async_copy(
                ps[b].at[pl.ds(p * DIM, DIM), pl.ds(0, SQRT)],
                out_hbm.at[p, :, pl.ds(lane0, SQRT)],
                sss[b]).wait()

    def transpose_p(u, b, p):
        pp0, _ = unit_pos(u)
        gb, pb = gs[b], ps[b]
        brow = [bias_v[pp0 + p, pl.ds(16 * dd, 16)]
                for dd in range(DIM // 16)]
        def srow(s, carry):
            sv0, sv1 = carry
            r = p * SQRT + 2 * s
            for h, sv in ((0, sv0), (1, sv1)):
                vals = [gb[r + h, pl.ds(16 * dd, 16)] + brow[dd]
                        for dd in range(DIM // 16)]
                for dd in range(DIM // 16):
                    plsc.store_scatter(pb, [pd_vecs[p][dd], sv], vals[dd])
            return (sv0 + 2, sv1 + 2)

        lax.fori_loop(0, SQRT // 2, srow,
                      (jnp.zeros((16,), jnp.int32),
                       jnp.full((16,), 1, jnp.int32)), unroll=4)

    for b in range(2):
        for k in range(GPU_):
            fire_gather(b, b, k)

    def body(tt, _):
        for b in range(2):
            u = 2 * tt + b
            drain_gather(b, 0)

            @pl.when(tt >= 1)
            def _ds():
                drain_stores(b)

            transpose_p(u, b, 0)
            fire_store(u, b, 0)

            @pl.when(u + 2 < N_UNITS)
            def _fg0():
                fire_gather(u + 2, b, 0)

            drain_gather(b, 1)
            transpose_p(u, b, 1)
            fire_store(u, b, 1)

            @pl.when(u + 2 < N_UNITS)
            def _fg1():
                fire_gather(u + 2, b, 1)

        return _

    lax.fori_loop(0, N_UNITS // 2, body, 0, unroll=False)
    drain_stores(0)
    drain_stores(1)


@jax.jit
def _run(xu, token_table, beat_table, bar_table):
    pos = jnp.asarray(_POS_NP)
    bias = pl.pallas_call(
        _bias_body,
        out_shape=jax.ShapeDtypeStruct((L, DIM), jnp.float32),
    )(pos, beat_table, bar_table)

    mesh = plsc.VectorSubcoreMesh(core_axis_name="c", subcore_axis_name="s")
    f = pl.kernel(
        _sc_body,
        out_type=jax.ShapeDtypeStruct((L, DIM, B), jnp.float32),
        mesh=mesh,
        scratch_types=[
            pltpu.VMEM((N_UNITS, GPU_, 128), jnp.int32),
            pltpu.VMEM((L, DIM), jnp.float32),
            pltpu.VMEM((ROWS_U, DIM), jnp.float32),
            pltpu.VMEM((ROWS_U, DIM), jnp.float32),
            pltpu.VMEM((PBLK * DIM, SQP), jnp.float32),
            pltpu.VMEM((PBLK * DIM, SQP), jnp.float32),
            pltpu.SemaphoreType.DMA,
            pltpu.SemaphoreType.DMA,
            pltpu.SemaphoreType.DMA,
            pltpu.SemaphoreType.DMA,
            pltpu.SemaphoreType.DMA,
            pltpu.SemaphoreType.DMA,
        ],
        compiler_params=pltpu.CompilerParams(use_tc_tiling_on_sc=False,
                                             needs_layout_passes=False),
        name="rhythm_embed_sc",
    )
    out_t = f(xu, token_table, bias)
    return jnp.transpose(out_t, (2, 0, 1))


def kernel(x, token_table, beat_table, bar_table):
    xi = x.astype(jnp.int32)
    xu = (xi.reshape(NW, SQRT, L // PBLK, PBLK)
          .transpose(0, 2, 3, 1)
          .reshape(NW, N_UNITS, GPU_, 128))
    return _run(xu, token_table, beat_table, bar_table)


# speedup vs baseline: 1.6254x; 1.6201x over previous
import functools

import jax
import jax.numpy as jnp
import numpy as np
from jax import lax
from jax.experimental import pallas as pl
from jax.experimental.pallas import tpu as pltpu
from jax.experimental.pallas import tpu_sc as plsc

VOCAB = 100000
DIM = 64
B = 4096
L = 200

_info = plsc.get_sparse_core_info()
NC, NS = _info.num_cores, _info.num_subcores
NW = NC * NS
SEQ_W = B // NW
PBLK = 2
SQRT = 128
NQ_U = SEQ_W // SQRT
N_UNITS = (L // PBLK) * NQ_U
ROWS_U = PBLK * SQRT
GPU_ = ROWS_U // 128
SQP = SQRT + 1


def _pos_encoding_np():
    positions = np.arange(L)[:, np.newaxis].astype(np.float64)
    dims = np.arange(DIM)[np.newaxis, :].astype(np.float64)
    angles = positions / np.power(10000.0, 2 * (dims // 2) / DIM)
    angles[:, 0::2] = np.sin(angles[:, 0::2])
    angles[:, 1::2] = np.cos(angles[:, 1::2])
    return angles.astype(np.float32)


_POS_NP = _pos_encoding_np()


def _bias_body(pos_ref, beat_ref, bar_ref, out_ref):
    beat = jnp.tile(beat_ref[...], (L // 4, 1))
    bar = jnp.tile(bar_ref[...], (L // 16 + 1, 1))[:L]
    out_ref[...] = pos_ref[...] + beat + bar


def _sc_body(xu_hbm, tok_hbm, bias_hbm, out_hbm,
             idx_v, bias_v, g0, g1, p0b, p1b,
             sg00, sg01, sg10, sg11, ss0, ss1):
    cid = lax.axis_index("c")
    sid = lax.axis_index("s")
    wid = sid * NC + cid
    lane0 = wid * SEQ_W

    gs = (g0, g1)
    ps = (p0b, p1b)
    sgs = ((sg00, sg01), (sg10, sg11))
    sss = (ss0, ss1)

    pltpu.sync_copy(xu_hbm.at[wid], idx_v)
    pltpu.sync_copy(bias_hbm, bias_v)

    pd_vecs = [[lax.iota(jnp.int32, 16) + (p * DIM + 16 * dd)
                for dd in range(DIM // 16)] for p in range(PBLK)]

    def unit_pos(u):
        return lax.div(u, NQ_U) * PBLK, lax.rem(u, NQ_U) * SQRT

    def fire_gather(u, b, k):
        pltpu.async_copy(
            tok_hbm.at[idx_v.at[u, k]],
            gs[b].at[pl.ds(k * 128, 128)],
            sgs[b][k])

    def drain_gather(b, k):
        pltpu.make_async_copy(
            tok_hbm.at[idx_v.at[0, k]],
            gs[b].at[pl.ds(k * 128, 128)], sgs[b][k]).wait()

    def fire_store(u, b, p):
        pp0, _ = unit_pos(u)
        for dt in range(DIM // 8):
            pltpu.async_copy(
                ps[b].at[pl.ds(p * DIM + dt * 8, 8), pl.ds(0, SQRT)],
                out_hbm.at[pp0 + p, dt, wid],
                sss[b])

    def drain_stores(b):
        for p in range(PBLK):
            for dt in range(DIM // 8):
                pltpu.make_async_copy(
                    ps[b].at[pl.ds(p * DIM + dt * 8, 8), pl.ds(0, SQRT)],
                    out_hbm.at[p, dt, wid],
                    sss[b]).wait()

    def transpose_p(u, b, p):
        pp0, _ = unit_pos(u)
        gb, pb = gs[b], ps[b]
        brow = [bias_v[pp0 + p, pl.ds(16 * dd, 16)]
                for dd in range(DIM // 16)]
        def srow(s, carry):
            sv0, sv1 = carry
            r = p * SQRT + 2 * s
            for h, sv in ((0, sv0), (1, sv1)):
                vals = [gb[r + h, pl.ds(16 * dd, 16)] + brow[dd]
                        for dd in range(DIM // 16)]
                for dd in range(DIM // 16):
                    plsc.store_scatter(pb, [pd_vecs[p][dd], sv], vals[dd])
            return (sv0 + 2, sv1 + 2)

        lax.fori_loop(0, SQRT // 2, srow,
                      (jnp.zeros((16,), jnp.int32),
                       jnp.full((16,), 1, jnp.int32)), unroll=2)

    for b in range(2):
        for k in range(GPU_):
            fire_gather(b, b, k)

    def body(tt, _):
        for b in range(2):
            u = 2 * tt + b
            drain_gather(b, 0)

            @pl.when(tt >= 1)
            def _ds():
                drain_stores(b)

            transpose_p(u, b, 0)
            fire_store(u, b, 0)

            @pl.when(u + 2 < N_UNITS)
            def _fg0():
                fire_gather(u + 2, b, 0)

            drain_gather(b, 1)
            transpose_p(u, b, 1)
            fire_store(u, b, 1)

            @pl.when(u + 2 < N_UNITS)
            def _fg1():
                fire_gather(u + 2, b, 1)

        return _

    lax.fori_loop(0, N_UNITS // 2, body, 0, unroll=False)
    drain_stores(0)
    drain_stores(1)


@jax.jit
def _run(xu, token_table, beat_table, bar_table):
    pos = jnp.asarray(_POS_NP)
    bias = pl.pallas_call(
        _bias_body,
        out_shape=jax.ShapeDtypeStruct((L, DIM), jnp.float32),
    )(pos, beat_table, bar_table)

    mesh = plsc.VectorSubcoreMesh(core_axis_name="c", subcore_axis_name="s")
    f = pl.kernel(
        _sc_body,
        out_type=jax.ShapeDtypeStruct((L, DIM // 8, NW, 8, SQRT), jnp.float32),
        mesh=mesh,
        scratch_types=[
            pltpu.VMEM((N_UNITS, GPU_, 128), jnp.int32),
            pltpu.VMEM((L, DIM), jnp.float32),
            pltpu.VMEM((ROWS_U, DIM), jnp.float32),
            pltpu.VMEM((ROWS_U, DIM), jnp.float32),
            pltpu.VMEM((PBLK * DIM, SQP), jnp.float32),
            pltpu.VMEM((PBLK * DIM, SQP), jnp.float32),
            pltpu.SemaphoreType.DMA,
            pltpu.SemaphoreType.DMA,
            pltpu.SemaphoreType.DMA,
            pltpu.SemaphoreType.DMA,
            pltpu.SemaphoreType.DMA,
            pltpu.SemaphoreType.DMA,
        ],
        compiler_params=pltpu.CompilerParams(use_tc_tiling_on_sc=False,
                                             needs_layout_passes=False),
        name="rhythm_embed_sc",
    )
    out5 = f(xu, token_table, bias)
    return jnp.transpose(out5, (2, 4, 0, 1, 3)).reshape(B, L, DIM)


def kernel(x, token_table, beat_table, bar_table):
    xi = x.astype(jnp.int32)
    xu = (xi.reshape(NW, SQRT, L // PBLK, PBLK)
          .transpose(0, 2, 3, 1)
          .reshape(NW, N_UNITS, GPU_, 128))
    return _run(xu, token_table, beat_table, bar_table)
